# Initial kernel scaffold; baseline (speedup 1.0000x reference)
import jax
import jax.numpy as jnp
from jax.experimental import pallas as pl


def kernel(edge_index, meshfield, W0, b0, p_w, W1, b1, W2, b2):
    raise NotImplementedError("write your pallas kernel here")


# speedup vs baseline: 4.6908x; 4.6908x over previous
import functools

import numpy as np
import jax
import jax.numpy as jnp
from jax import lax
from jax.experimental import pallas as pl
from jax.experimental.pallas import tpu as pltpu
from jax.experimental.pallas import tpu_sc as plsc

N_T = 4
N_F = 3
POOL_RATIO = 0.5

NC = 2
NS = 16
NW = NC * NS
CHUNK = 128
_IP = False


def _cdiv(a, b):
    return (a + b - 1) // b


def _sc_segsum(table, gidx, sidx, NP, CH_W, C):
    NPT = NP // NS
    mesh = plsc.VectorSubcoreMesh(core_axis_name="c", subcore_axis_name="s",
                                  num_cores=NC, num_subcores=NS)

    @functools.partial(
        pl.kernel,
        interpret=_IP,
        compiler_params=pltpu.CompilerParams(use_tc_tiling_on_sc=False),
        out_type=jax.ShapeDtypeStruct((NC, NP, C), jnp.float32),
        mesh=mesh,
        scratch_types=[
            pltpu.VMEM((CH_W, CHUNK), jnp.int32),
            pltpu.VMEM((CH_W, CHUNK), jnp.int32),
            pltpu.VMEM((CHUNK, C), jnp.float32),
            pltpu.VMEM((NPT, C), jnp.float32),
            pltpu.VMEM_SHARED((NP, C), jnp.float32),
            pltpu.SemaphoreType.DMA,
        ],
    )
    def kfn(table_hbm, gidx_hbm, sidx_hbm, zeros_hbm, out_hbm,
            gp, sp, rows, obuf, acc, sem):
        cid = lax.axis_index("c")
        sid = lax.axis_index("s")
        wid = cid * NS + sid
        pltpu.sync_copy(zeros_hbm.at[pl.ds(sid * NPT, NPT)],
                        acc.at[pl.ds(sid * NPT, NPT)])
        pltpu.sync_copy(gidx_hbm.at[wid], gp)
        pltpu.sync_copy(sidx_hbm.at[wid], sp)
        plsc.subcore_barrier()

        def body(j, carry):
            pltpu.async_copy(table_hbm.at[gp.at[j]], rows, sem).wait()
            pltpu.sync_copy(rows, acc.at[sp.at[j]], add=True)
            return carry

        lax.fori_loop(0, CH_W, body, 0)
        plsc.subcore_barrier()
        pltpu.sync_copy(acc.at[pl.ds(sid * NPT, NPT)], obuf)
        pltpu.sync_copy(obuf, out_hbm.at[cid, pl.ds(sid * NPT, NPT)])

    zeros = jnp.zeros((NP, C), jnp.float32)
    return kfn(table, gidx, sidx, zeros)


def _sc_degd2(scolp, growp, rcnt8, NP, CH_W):
    NPT = NP // NS
    C = 8
    mesh = plsc.VectorSubcoreMesh(core_axis_name="c", subcore_axis_name="s",
                                  num_cores=NC, num_subcores=NS)

    @functools.partial(
        pl.kernel,
        interpret=_IP,
        compiler_params=pltpu.CompilerParams(use_tc_tiling_on_sc=False),
        out_type=jax.ShapeDtypeStruct((NC, NP, C), jnp.float32),
        mesh=mesh,
        scratch_types=[
            pltpu.VMEM((CH_W, CHUNK), jnp.int32),
            pltpu.VMEM((CH_W, CHUNK), jnp.int32),
            pltpu.VMEM((CH_W, CHUNK, C), jnp.float32),
            pltpu.VMEM((CHUNK, C), jnp.float32),
            pltpu.VMEM((NPT, C), jnp.float32),
            pltpu.VMEM_SHARED((NP, C), jnp.float32),
        ],
    )
    def kfn(scolp_hbm, growp_hbm, rcnt_hbm, ones_hbm, zeros_hbm, out_hbm,
            sp, gp, rp, ones_v, obuf, acc):
        cid = lax.axis_index("c")
        sid = lax.axis_index("s")
        wid = cid * NS + sid
        pltpu.sync_copy(zeros_hbm.at[pl.ds(sid * NPT, NPT)],
                        acc.at[pl.ds(sid * NPT, NPT)])
        pltpu.sync_copy(scolp_hbm.at[wid], sp)
        pltpu.sync_copy(growp_hbm.at[wid], gp)
        pltpu.sync_copy(rcnt_hbm.at[wid], rp)
        pltpu.sync_copy(ones_hbm, ones_v)
        plsc.subcore_barrier()

        def body(j, carry):
            pltpu.sync_copy(ones_v, acc.at[sp.at[j]], add=True)
            pltpu.sync_copy(rp.at[j], acc.at[gp.at[j]], add=True)
            return carry

        lax.fori_loop(0, CH_W, body, 0)
        plsc.subcore_barrier()
        pltpu.sync_copy(acc.at[pl.ds(sid * NPT, NPT)], obuf)
        pltpu.sync_copy(obuf, out_hbm.at[cid, pl.ds(sid * NPT, NPT)])

    ones = jnp.concatenate(
        [jnp.ones((CHUNK, 1), jnp.float32),
         jnp.zeros((CHUNK, C - 1), jnp.float32)], 1)
    zeros = jnp.zeros((NP, C), jnp.float32)
    return kfn(scolp, growp, rcnt8, ones, zeros)


def _tc_call(fn, out_shape):
    return pl.pallas_call(fn, out_shape=out_shape, interpret=_IP)


def _tca(meshfield_p, W0, parts, N, NP):
    def fn(mesh_ref, w0_ref, parts_ref, h0_ref, h0s_ref, pk0_ref):
        h0 = jnp.dot(mesh_ref[...], w0_ref[...],
                     preferred_element_type=jnp.float32)
        indeg = parts_ref[0, :, 0:1] + parts_ref[1, :, 0:1]
        dinv0 = lax.rsqrt(indeg + 2.0)
        d2 = parts_ref[0, :, 1:2] + parts_ref[1, :, 1:2]
        h0_ref[...] = h0
        h0s_ref[...] = h0 * dinv0
        pk0_ref[...] = jnp.concatenate(
            [dinv0, d2, jnp.zeros((NP, 6), jnp.float32)], axis=1)

    H = W0.shape[1]
    return _tc_call(fn, [
        jax.ShapeDtypeStruct((NP, H), jnp.float32),
        jax.ShapeDtypeStruct((NP, H), jnp.float32),
        jax.ShapeDtypeStruct((NP, 8), jnp.float32),
    ])(meshfield_p, W0, parts)


def _tcc(m0p, h0, pk0, b0r, pwc, N, NP):
    def fn(m0p_ref, h0_ref, pk0_ref, b0_ref, pw_ref, x0_ref, score_ref):
        m0 = m0p_ref[0] + m0p_ref[1]
        dinv0 = pk0_ref[:, 0:1]
        h0 = h0_ref[...]
        pre = dinv0 * m0 + 2.0 * dinv0 * dinv0 * h0 + b0_ref[...]
        mask = lax.broadcasted_iota(jnp.int32, pre.shape, 0) < N
        x0 = jnp.where(mask, jnp.maximum(pre, 0.0), 0.0)
        pw = pw_ref[...]
        pwn = pw * lax.rsqrt(jnp.sum(pw * pw))
        score = jnp.tanh(jnp.dot(x0, pwn, preferred_element_type=jnp.float32))
        x0_ref[...] = x0
        score_ref[...] = score

    H = h0.shape[1]
    return _tc_call(fn, [
        jax.ShapeDtypeStruct((NP, H), jnp.float32),
        jax.ShapeDtypeStruct((NP, 1), jnp.float32),
    ])(m0p, h0, pk0, b0r, pwc)


def _tck(score2d, k):
    R = score2d.shape[0]

    def fn(s_ref, sel_ref):
        s = s_ref[...]
        sb = lax.bitcast_convert_type(s, jnp.int32)
        u = lax.bitcast_convert_type(s, jnp.uint32)
        ukey = jnp.where(sb < 0, ~u, u | jnp.uint32(0x80000000))
        T = jnp.uint32(0)
        for bit in range(31, -1, -1):
            cand = T | jnp.uint32(1 << bit)
            cnt = jnp.sum((ukey >= cand).astype(jnp.float32))
            T = jnp.where(cnt >= k, cand, T)
        n_gt = jnp.sum((ukey > T).astype(jnp.float32))
        need = jnp.float32(k) - n_gt
        tie = ukey == T
        idx = (lax.broadcasted_iota(jnp.int32, s.shape, 0) * 128
               + lax.broadcasted_iota(jnp.int32, s.shape, 1))
        X = jnp.int32(0)
        for bit in range(14, -1, -1):
            cand = X | jnp.int32(1 << bit)
            g = jnp.sum((tie & (idx < cand)).astype(jnp.float32))
            X = jnp.where(g < need, cand, X)
        sel = (ukey > T) | (tie & (idx <= X) & (need > 0))
        sel_ref[...] = sel.astype(jnp.float32)

    return _tc_call(fn, jax.ShapeDtypeStruct((R, 128), jnp.float32))(score2d)


def _tcy(x0, score, sel, W1, N, NP):
    def fn(x0_ref, sc_ref, sel_ref, w1_ref, h1n_ref, sel8_ref):
        y0 = x0_ref[...] * (sc_ref[...] * sel_ref[...])
        h1n_ref[...] = jnp.dot(y0, w1_ref[...],
                               preferred_element_type=jnp.float32)
        sel8_ref[...] = jnp.concatenate(
            [sel_ref[...], jnp.zeros((NP, 7), jnp.float32)], axis=1)

    H = W1.shape[1]
    return _tc_call(fn, [
        jax.ShapeDtypeStruct((NP, H), jnp.float32),
        jax.ShapeDtypeStruct((NP, 8), jnp.float32),
    ])(x0, score, sel, W1)


def _tcu1(sel, u1p, N, NP):
    def fn(sel_ref, u1p_ref, u18_ref):
        u1 = sel_ref[...] + u1p_ref[0, :, 0:1] + u1p_ref[1, :, 0:1]
        mask = lax.broadcasted_iota(jnp.int32, (NP, 1), 0) < N
        u1 = jnp.where(mask, u1, 0.0)
        u18_ref[...] = jnp.concatenate(
            [u1, jnp.zeros((NP, 7), jnp.float32)], axis=1)

    return _tc_call(fn, jax.ShapeDtypeStruct((NP, 8), jnp.float32))(sel, u1p)


def _tcu2(u18, u2p, pk0, sel, h1n, N, NP):
    def fn(u18_ref, u2p_ref, pk0_ref, sel_ref, h1n_ref, z_ref, pk1_ref):
        u2 = u18_ref[:, 0:1] + u2p_ref[0, :, 0:1] + u2p_ref[1, :, 0:1]
        sel = sel_ref[...]
        d2 = pk0_ref[:, 1:2]
        degp = u2 - (1.0 + d2) * sel + 2.0
        dinvp = lax.rsqrt(degp)
        z_ref[...] = sel * dinvp * h1n_ref[...]
        pk1_ref[...] = jnp.concatenate(
            [dinvp, sel, jnp.zeros((NP, 6), jnp.float32)], axis=1)

    H = h1n.shape[1]
    return _tc_call(fn, [
        jax.ShapeDtypeStruct((NP, H), jnp.float32),
        jax.ShapeDtypeStruct((NP, 8), jnp.float32),
    ])(u18, u2p, pk0, sel, h1n)


def _tcw(Z, wp, N, NP):
    def fn(z_ref, wp_ref, w_ref):
        w = z_ref[...] + wp_ref[0] + wp_ref[1]
        mask = lax.broadcasted_iota(jnp.int32, w.shape, 0) < N
        w_ref[...] = jnp.where(mask, w, 0.0)

    return _tc_call(fn, jax.ShapeDtypeStruct(Z.shape, jnp.float32))(Z, wp)


def _tcm(Wf, up, Z, pk0, N, NP):
    def fn(wf_ref, up_ref, z_ref, pk0_ref, msg_ref):
        U = wf_ref[...] + up_ref[0] + up_ref[1]
        d2 = pk0_ref[:, 1:2]
        msg_ref[...] = U - (1.0 + d2) * z_ref[...]

    return _tc_call(fn, jax.ShapeDtypeStruct(Z.shape, jnp.float32))(
        Wf, up, Z, pk0)


def _tcx(msg, pk1, h1n, x0, b1r, N, NP):
    def fn(msg_ref, pk1_ref, h1n_ref, x0_ref, b1_ref, xs_ref):
        dinvp = pk1_ref[:, 0:1]
        sel = pk1_ref[:, 1:2]
        h1n = h1n_ref[...]
        x1n = jnp.maximum(
            dinvp * msg_ref[...] + 2.0 * dinvp * dinvp * h1n + b1_ref[...],
            0.0)
        xs_ref[...] = x0_ref[...] + sel * x1n

    return _tc_call(fn, jax.ShapeDtypeStruct(msg.shape, jnp.float32))(
        msg, pk1, h1n, x0, b1r)


def _tce2(xs, W2p, pk0, N, NP):
    def fn(xs_ref, w2_ref, pk0_ref, h2_ref, h2s_ref):
        h2 = jnp.dot(xs_ref[...], w2_ref[...],
                     preferred_element_type=jnp.float32)
        h2_ref[...] = h2
        h2s_ref[...] = h2 * pk0_ref[:, 0:1]

    C2 = W2p.shape[1]
    return _tc_call(fn, [
        jax.ShapeDtypeStruct((NP, C2), jnp.float32),
        jax.ShapeDtypeStruct((NP, C2), jnp.float32),
    ])(xs, W2p, pk0)


def _tcf(m2p, h2, pk0, b2r, N, NP):
    def fn(m2p_ref, h2_ref, pk0_ref, b2_ref, out_ref):
        m2 = m2p_ref[0] + m2p_ref[1]
        dinv0 = pk0_ref[:, 0:1]
        out_ref[...] = (dinv0 * m2 + 2.0 * dinv0 * dinv0 * h2_ref[...]
                        + b2_ref[...])

    return _tc_call(fn, jax.ShapeDtypeStruct(h2.shape, jnp.float32))(
        m2p, h2, pk0, b2r)


def kernel(edge_index, meshfield, W0, b0, p_w, W1, b1, W2, b2):
    N, IN_CH = meshfield.shape
    E = edge_index.shape[1]
    HID = W0.shape[1]
    OUT_CH = W2.shape[1]
    k = int(np.ceil(POOL_RATIO * N))
    NP = _cdiv(N + 1, NS * 8) * NS * 8
    DUMMY = N
    CH_W = _cdiv(E, NW * CHUNK)
    E_pad = NW * CH_W * CHUNK

    row = edge_index[0].astype(jnp.int32)
    col = edge_index[1].astype(jnp.int32)
    nz = row != col

    key = row * N + col
    BIG = jnp.int32(N * N)
    skeys = jnp.sort(jnp.where(nz, key, BIG))
    rkey = col * N + row
    lo = jnp.searchsorted(skeys, rkey, side='left')
    hi = jnp.searchsorted(skeys, rkey, side='right')
    rcount = (hi - lo).astype(jnp.float32) * nz.astype(jnp.float32)

    colm = jnp.where(nz, col, DUMMY)

    def _plane(x, fill):
        xp = jnp.concatenate(
            [x, jnp.full((E_pad - E,), fill, x.dtype)])
        return xp.reshape(NW, CH_W, CHUNK)

    growp = _plane(row, DUMMY)
    scolp = _plane(colm, DUMMY)
    rcnt8 = jnp.pad(
        jnp.concatenate([rcount, jnp.zeros((E_pad - E,), jnp.float32)])[:, None],
        ((0, 0), (1, 6))).reshape(NW, CH_W, CHUNK, 8)

    meshfield_p = jnp.pad(meshfield, ((0, NP - N), (0, 0)))
    b0r = b0[None, :]
    b1r = b1[None, :]
    W2p = jnp.pad(W2, ((0, 0), (0, 16 - OUT_CH)))
    b2r = jnp.pad(b2, (0, 16 - OUT_CH))[None, :]
    pwc = p_w[:, None]

    parts = _sc_degd2(scolp, growp, rcnt8, NP, CH_W)
    h0, h0s, pk0 = _tca(meshfield_p, W0, parts, N, NP)
    m0p = _sc_segsum(h0s, growp, scolp, NP, CH_W, HID)
    x0, score = _tcc(m0p, h0, pk0, b0r, pwc, N, NP)
    NPAD2 = _cdiv(N, 1024) * 1024
    sflat = score[:N, 0]
    s2d = jnp.concatenate(
        [sflat, jnp.full((NPAD2 - N,), -jnp.inf, jnp.float32)]).reshape(-1, 128)
    sel2d = _tck(s2d, k)
    sel = sel2d.reshape(-1)[:N][:, None]
    sel = jnp.pad(sel, ((0, NP - N), (0, 0)))
    h1n, sel8 = _tcy(x0, score, sel, W1, N, NP)
    u1p = _sc_segsum(sel8, growp, scolp, NP, CH_W, 8)
    u18 = _tcu1(sel, u1p, N, NP)
    u2p = _sc_segsum(u18, growp, scolp, NP, CH_W, 8)
    Z, pk1 = _tcu2(u18, u2p, pk0, sel, h1n, N, NP)
    wp = _sc_segsum(Z, growp, scolp, NP, CH_W, HID)
    Wf = _tcw(Z, wp, N, NP)
    up = _sc_segsum(Wf, growp, scolp, NP, CH_W, HID)
    msg = _tcm(Wf, up, Z, pk0, N, NP)
    xs = _tcx(msg, pk1, h1n, x0, b1r, N, NP)
    h2, h2s = _tce2(xs, W2p, pk0, N, NP)
    m2p = _sc_segsum(h2s, growp, scolp, NP, CH_W, 16)
    out16 = _tcf(m2p, h2, pk0, b2r, N, NP)
    return out16[:N, :OUT_CH].reshape(N, N_T, N_F)


# speedup vs baseline: 4.7407x; 1.0106x over previous
import functools

import numpy as np
import jax
import jax.numpy as jnp
from jax import lax
from jax.experimental import pallas as pl
from jax.experimental.pallas import tpu as pltpu
from jax.experimental.pallas import tpu_sc as plsc

N_T = 4
N_F = 3
POOL_RATIO = 0.5

NC = 2
NS = 16
NW = NC * NS
CHUNK = 128
_IP = False


def _cdiv(a, b):
    return (a + b - 1) // b


def _sc_segsum(table, gidx, sidx, NP, CH_W, C):
    NPT = NP // NS
    mesh = plsc.VectorSubcoreMesh(core_axis_name="c", subcore_axis_name="s",
                                  num_cores=NC, num_subcores=NS)

    @functools.partial(
        pl.kernel,
        interpret=_IP,
        compiler_params=pltpu.CompilerParams(use_tc_tiling_on_sc=False),
        out_type=jax.ShapeDtypeStruct((NC, NP, C), jnp.float32),
        mesh=mesh,
        scratch_types=[
            pltpu.VMEM((CH_W, CHUNK), jnp.int32),
            pltpu.VMEM((CH_W, CHUNK), jnp.int32),
            pltpu.VMEM((CHUNK, C), jnp.float32),
            pltpu.VMEM((CHUNK, C), jnp.float32),
            pltpu.VMEM((NPT, C), jnp.float32),
            pltpu.VMEM_SHARED((NP, C), jnp.float32),
            pltpu.SemaphoreType.DMA,
            pltpu.SemaphoreType.DMA,
        ],
    )
    def kfn(table_hbm, gidx_hbm, sidx_hbm, zeros_hbm, out_hbm,
            gp, sp, rows0, rows1, obuf, acc, sem0, sem1):
        cid = lax.axis_index("c")
        sid = lax.axis_index("s")
        wid = cid * NS + sid
        pltpu.sync_copy(zeros_hbm.at[pl.ds(sid * NPT, NPT)],
                        acc.at[pl.ds(sid * NPT, NPT)])
        pltpu.sync_copy(gidx_hbm.at[wid], gp)
        pltpu.sync_copy(sidx_hbm.at[wid], sp)
        plsc.subcore_barrier()
        pltpu.async_copy(table_hbm.at[gp.at[0]], rows0, sem0)

        def body(i, carry):
            j0 = 2 * i
            j1 = 2 * i + 1
            pltpu.make_async_copy(table_hbm.at[gp.at[j0]], rows0, sem0).wait()
            pltpu.async_copy(table_hbm.at[gp.at[j1]], rows1, sem1)
            pltpu.sync_copy(rows0, acc.at[sp.at[j0]], add=True)
            pltpu.make_async_copy(table_hbm.at[gp.at[j1]], rows1, sem1).wait()

            @pl.when(j1 + 1 < CH_W)
            def _():
                pltpu.async_copy(table_hbm.at[gp.at[j1 + 1]], rows0, sem0)

            pltpu.sync_copy(rows1, acc.at[sp.at[j1]], add=True)
            return carry

        lax.fori_loop(0, CH_W // 2, body, 0)
        plsc.subcore_barrier()
        pltpu.sync_copy(acc.at[pl.ds(sid * NPT, NPT)], obuf)
        pltpu.sync_copy(obuf, out_hbm.at[cid, pl.ds(sid * NPT, NPT)])

    zeros = jnp.zeros((NP, C), jnp.float32)
    return kfn(table, gidx, sidx, zeros)


def _sc_degd2(scolp, growp, rcnt8, NP, CH_W):
    NPT = NP // NS
    C = 8
    mesh = plsc.VectorSubcoreMesh(core_axis_name="c", subcore_axis_name="s",
                                  num_cores=NC, num_subcores=NS)

    @functools.partial(
        pl.kernel,
        interpret=_IP,
        compiler_params=pltpu.CompilerParams(use_tc_tiling_on_sc=False),
        out_type=jax.ShapeDtypeStruct((NC, NP, C), jnp.float32),
        mesh=mesh,
        scratch_types=[
            pltpu.VMEM((CH_W, CHUNK), jnp.int32),
            pltpu.VMEM((CH_W, CHUNK), jnp.int32),
            pltpu.VMEM((CH_W, CHUNK, C), jnp.float32),
            pltpu.VMEM((CHUNK, C), jnp.float32),
            pltpu.VMEM((NPT, C), jnp.float32),
            pltpu.VMEM_SHARED((NP, C), jnp.float32),
        ],
    )
    def kfn(scolp_hbm, growp_hbm, rcnt_hbm, ones_hbm, zeros_hbm, out_hbm,
            sp, gp, rp, ones_v, obuf, acc):
        cid = lax.axis_index("c")
        sid = lax.axis_index("s")
        wid = cid * NS + sid
        pltpu.sync_copy(zeros_hbm.at[pl.ds(sid * NPT, NPT)],
                        acc.at[pl.ds(sid * NPT, NPT)])
        pltpu.sync_copy(scolp_hbm.at[wid], sp)
        pltpu.sync_copy(growp_hbm.at[wid], gp)
        pltpu.sync_copy(rcnt_hbm.at[wid], rp)
        pltpu.sync_copy(ones_hbm, ones_v)
        plsc.subcore_barrier()

        def body(j, carry):
            pltpu.sync_copy(ones_v, acc.at[sp.at[j]], add=True)
            pltpu.sync_copy(rp.at[j], acc.at[gp.at[j]], add=True)
            return carry

        lax.fori_loop(0, CH_W, body, 0)
        plsc.subcore_barrier()
        pltpu.sync_copy(acc.at[pl.ds(sid * NPT, NPT)], obuf)
        pltpu.sync_copy(obuf, out_hbm.at[cid, pl.ds(sid * NPT, NPT)])

    ones = jnp.concatenate(
        [jnp.ones((CHUNK, 1), jnp.float32),
         jnp.zeros((CHUNK, C - 1), jnp.float32)], 1)
    zeros = jnp.zeros((NP, C), jnp.float32)
    return kfn(scolp, growp, rcnt8, ones, zeros)


def _tc_call(fn, out_shape):
    return pl.pallas_call(fn, out_shape=out_shape, interpret=_IP)


def _tca(meshfield_p, W0, parts, N, NP):
    def fn(mesh_ref, w0_ref, parts_ref, h0_ref, h0s_ref, pk0_ref):
        h0 = jnp.dot(mesh_ref[...], w0_ref[...],
                     preferred_element_type=jnp.float32)
        indeg = parts_ref[0, :, 0:1] + parts_ref[1, :, 0:1]
        dinv0 = lax.rsqrt(indeg + 2.0)
        d2 = parts_ref[0, :, 1:2] + parts_ref[1, :, 1:2]
        h0_ref[...] = h0
        h0s_ref[...] = h0 * dinv0
        pk0_ref[...] = jnp.concatenate(
            [dinv0, d2, jnp.zeros((NP, 6), jnp.float32)], axis=1)

    H = W0.shape[1]
    return _tc_call(fn, [
        jax.ShapeDtypeStruct((NP, H), jnp.float32),
        jax.ShapeDtypeStruct((NP, H), jnp.float32),
        jax.ShapeDtypeStruct((NP, 8), jnp.float32),
    ])(meshfield_p, W0, parts)


def _tcc(m0p, h0, pk0, b0r, pwc, N, NP):
    def fn(m0p_ref, h0_ref, pk0_ref, b0_ref, pw_ref, x0_ref, score_ref):
        m0 = m0p_ref[0] + m0p_ref[1]
        dinv0 = pk0_ref[:, 0:1]
        h0 = h0_ref[...]
        pre = dinv0 * m0 + 2.0 * dinv0 * dinv0 * h0 + b0_ref[...]
        mask = lax.broadcasted_iota(jnp.int32, pre.shape, 0) < N
        x0 = jnp.where(mask, jnp.maximum(pre, 0.0), 0.0)
        pw = pw_ref[...]
        pwn = pw * lax.rsqrt(jnp.sum(pw * pw))
        score = jnp.tanh(jnp.dot(x0, pwn, preferred_element_type=jnp.float32))
        x0_ref[...] = x0
        score_ref[...] = score

    H = h0.shape[1]
    return _tc_call(fn, [
        jax.ShapeDtypeStruct((NP, H), jnp.float32),
        jax.ShapeDtypeStruct((NP, 1), jnp.float32),
    ])(m0p, h0, pk0, b0r, pwc)


def _tck(score2d, k):
    R = score2d.shape[0]

    def fn(s_ref, sel_ref):
        s = s_ref[...]
        sb = lax.bitcast_convert_type(s, jnp.int32)
        u = lax.bitcast_convert_type(s, jnp.uint32)
        ukey = jnp.where(sb < 0, ~u, u | jnp.uint32(0x80000000))
        T = jnp.uint32(0)
        for bit in range(31, -1, -1):
            cand = T | jnp.uint32(1 << bit)
            cnt = jnp.sum((ukey >= cand).astype(jnp.float32))
            T = jnp.where(cnt >= k, cand, T)
        n_gt = jnp.sum((ukey > T).astype(jnp.float32))
        need = jnp.float32(k) - n_gt
        tie = ukey == T
        idx = (lax.broadcasted_iota(jnp.int32, s.shape, 0) * 128
               + lax.broadcasted_iota(jnp.int32, s.shape, 1))
        X = jnp.int32(0)
        for bit in range(14, -1, -1):
            cand = X | jnp.int32(1 << bit)
            g = jnp.sum((tie & (idx < cand)).astype(jnp.float32))
            X = jnp.where(g < need, cand, X)
        sel = (ukey > T) | (tie & (idx <= X) & (need > 0))
        sel_ref[...] = sel.astype(jnp.float32)

    return _tc_call(fn, jax.ShapeDtypeStruct((R, 128), jnp.float32))(score2d)


def _tcy(x0, score, sel, W1, N, NP):
    def fn(x0_ref, sc_ref, sel_ref, w1_ref, h1n_ref, sel8_ref):
        y0 = x0_ref[...] * (sc_ref[...] * sel_ref[...])
        h1n_ref[...] = jnp.dot(y0, w1_ref[...],
                               preferred_element_type=jnp.float32)
        sel8_ref[...] = jnp.concatenate(
            [sel_ref[...], jnp.zeros((NP, 7), jnp.float32)], axis=1)

    H = W1.shape[1]
    return _tc_call(fn, [
        jax.ShapeDtypeStruct((NP, H), jnp.float32),
        jax.ShapeDtypeStruct((NP, 8), jnp.float32),
    ])(x0, score, sel, W1)


def _tcu1(sel, u1p, N, NP):
    def fn(sel_ref, u1p_ref, u18_ref):
        u1 = sel_ref[...] + u1p_ref[0, :, 0:1] + u1p_ref[1, :, 0:1]
        mask = lax.broadcasted_iota(jnp.int32, (NP, 1), 0) < N
        u1 = jnp.where(mask, u1, 0.0)
        u18_ref[...] = jnp.concatenate(
            [u1, jnp.zeros((NP, 7), jnp.float32)], axis=1)

    return _tc_call(fn, jax.ShapeDtypeStruct((NP, 8), jnp.float32))(sel, u1p)


def _tcu2(u18, u2p, pk0, sel, h1n, N, NP):
    def fn(u18_ref, u2p_ref, pk0_ref, sel_ref, h1n_ref, z_ref, pk1_ref):
        u2 = u18_ref[:, 0:1] + u2p_ref[0, :, 0:1] + u2p_ref[1, :, 0:1]
        sel = sel_ref[...]
        d2 = pk0_ref[:, 1:2]
        degp = u2 - (1.0 + d2) * sel + 2.0
        dinvp = lax.rsqrt(degp)
        z_ref[...] = sel * dinvp * h1n_ref[...]
        pk1_ref[...] = jnp.concatenate(
            [dinvp, sel, jnp.zeros((NP, 6), jnp.float32)], axis=1)

    H = h1n.shape[1]
    return _tc_call(fn, [
        jax.ShapeDtypeStruct((NP, H), jnp.float32),
        jax.ShapeDtypeStruct((NP, 8), jnp.float32),
    ])(u18, u2p, pk0, sel, h1n)


def _tcw(Z, wp, N, NP):
    def fn(z_ref, wp_ref, w_ref):
        w = z_ref[...] + wp_ref[0] + wp_ref[1]
        mask = lax.broadcasted_iota(jnp.int32, w.shape, 0) < N
        w_ref[...] = jnp.where(mask, w, 0.0)

    return _tc_call(fn, jax.ShapeDtypeStruct(Z.shape, jnp.float32))(Z, wp)


def _tcm(Wf, up, Z, pk0, N, NP):
    def fn(wf_ref, up_ref, z_ref, pk0_ref, msg_ref):
        U = wf_ref[...] + up_ref[0] + up_ref[1]
        d2 = pk0_ref[:, 1:2]
        msg_ref[...] = U - (1.0 + d2) * z_ref[...]

    return _tc_call(fn, jax.ShapeDtypeStruct(Z.shape, jnp.float32))(
        Wf, up, Z, pk0)


def _tcx(msg, pk1, h1n, x0, b1r, N, NP):
    def fn(msg_ref, pk1_ref, h1n_ref, x0_ref, b1_ref, xs_ref):
        dinvp = pk1_ref[:, 0:1]
        sel = pk1_ref[:, 1:2]
        h1n = h1n_ref[...]
        x1n = jnp.maximum(
            dinvp * msg_ref[...] + 2.0 * dinvp * dinvp * h1n + b1_ref[...],
            0.0)
        xs_ref[...] = x0_ref[...] + sel * x1n

    return _tc_call(fn, jax.ShapeDtypeStruct(msg.shape, jnp.float32))(
        msg, pk1, h1n, x0, b1r)


def _tce2(xs, W2p, pk0, N, NP):
    def fn(xs_ref, w2_ref, pk0_ref, h2_ref, h2s_ref):
        h2 = jnp.dot(xs_ref[...], w2_ref[...],
                     preferred_element_type=jnp.float32)
        h2_ref[...] = h2
        h2s_ref[...] = h2 * pk0_ref[:, 0:1]

    C2 = W2p.shape[1]
    return _tc_call(fn, [
        jax.ShapeDtypeStruct((NP, C2), jnp.float32),
        jax.ShapeDtypeStruct((NP, C2), jnp.float32),
    ])(xs, W2p, pk0)


def _tcf(m2p, h2, pk0, b2r, N, NP):
    def fn(m2p_ref, h2_ref, pk0_ref, b2_ref, out_ref):
        m2 = m2p_ref[0] + m2p_ref[1]
        dinv0 = pk0_ref[:, 0:1]
        out_ref[...] = (dinv0 * m2 + 2.0 * dinv0 * dinv0 * h2_ref[...]
                        + b2_ref[...])

    return _tc_call(fn, jax.ShapeDtypeStruct(h2.shape, jnp.float32))(
        m2p, h2, pk0, b2r)


def kernel(edge_index, meshfield, W0, b0, p_w, W1, b1, W2, b2):
    N, IN_CH = meshfield.shape
    E = edge_index.shape[1]
    HID = W0.shape[1]
    OUT_CH = W2.shape[1]
    k = int(np.ceil(POOL_RATIO * N))
    NP = _cdiv(N + 1, NS * 8) * NS * 8
    DUMMY = N
    CH_W = 2 * _cdiv(E, NW * CHUNK * 2)
    E_pad = NW * CH_W * CHUNK

    row = edge_index[0].astype(jnp.int32)
    col = edge_index[1].astype(jnp.int32)
    nz = row != col

    key = row * N + col
    BIG = jnp.int32(N * N)
    skeys = jnp.sort(jnp.where(nz, key, BIG))
    rkey = col * N + row
    lo = jnp.searchsorted(skeys, rkey, side='left')
    hi = jnp.searchsorted(skeys, rkey, side='right')
    rcount = (hi - lo).astype(jnp.float32) * nz.astype(jnp.float32)

    colm = jnp.where(nz, col, DUMMY)

    def _plane(x, fill):
        xp = jnp.concatenate(
            [x, jnp.full((E_pad - E,), fill, x.dtype)])
        return xp.reshape(NW, CH_W, CHUNK)

    growp = _plane(row, DUMMY)
    scolp = _plane(colm, DUMMY)
    rcnt8 = jnp.pad(
        jnp.concatenate([rcount, jnp.zeros((E_pad - E,), jnp.float32)])[:, None],
        ((0, 0), (1, 6))).reshape(NW, CH_W, CHUNK, 8)

    meshfield_p = jnp.pad(meshfield, ((0, NP - N), (0, 0)))
    b0r = b0[None, :]
    b1r = b1[None, :]
    W2p = jnp.pad(W2, ((0, 0), (0, 16 - OUT_CH)))
    b2r = jnp.pad(b2, (0, 16 - OUT_CH))[None, :]
    pwc = p_w[:, None]

    parts = _sc_degd2(scolp, growp, rcnt8, NP, CH_W)
    h0, h0s, pk0 = _tca(meshfield_p, W0, parts, N, NP)
    m0p = _sc_segsum(h0s, growp, scolp, NP, CH_W, HID)
    x0, score = _tcc(m0p, h0, pk0, b0r, pwc, N, NP)
    NPAD2 = _cdiv(N, 1024) * 1024
    sflat = score[:N, 0]
    s2d = jnp.concatenate(
        [sflat, jnp.full((NPAD2 - N,), -jnp.inf, jnp.float32)]).reshape(-1, 128)
    sel2d = _tck(s2d, k)
    sel = sel2d.reshape(-1)[:N][:, None]
    sel = jnp.pad(sel, ((0, NP - N), (0, 0)))
    h1n, sel8 = _tcy(x0, score, sel, W1, N, NP)
    u1p = _sc_segsum(sel8, growp, scolp, NP, CH_W, 8)
    u18 = _tcu1(sel, u1p, N, NP)
    u2p = _sc_segsum(u18, growp, scolp, NP, CH_W, 8)
    Z, pk1 = _tcu2(u18, u2p, pk0, sel, h1n, N, NP)
    wp = _sc_segsum(Z, growp, scolp, NP, CH_W, HID)
    Wf = _tcw(Z, wp, N, NP)
    up = _sc_segsum(Wf, growp, scolp, NP, CH_W, HID)
    msg = _tcm(Wf, up, Z, pk0, N, NP)
    xs = _tcx(msg, pk1, h1n, x0, b1r, N, NP)
    h2, h2s = _tce2(xs, W2p, pk0, N, NP)
    m2p = _sc_segsum(h2s, growp, scolp, NP, CH_W, 16)
    out16 = _tcf(m2p, h2, pk0, b2r, N, NP)
    return out16[:N, :OUT_CH].reshape(N, N_T, N_F)
